# baseline (speedup 1.0000x reference)
import functools

import jax
import jax.numpy as jnp
from jax import lax
from jax.experimental import pallas as pl
from jax.experimental.pallas import tpu as pltpu
from jax.experimental.pallas import tpu_sc as plsc

NC = 2
NS = 16
CHUNK = 80
NSLOT = 4
GLEAD = 3
SDEFER = 1


def _segment_sum_sc(feat2, src2, dst3, n):
    dh = feat2.shape[1]
    nchunk = dst3.shape[1]
    npair = nchunk // 2
    zr = -(-n // (NS * 8)) * 8
    n_pad = zr * NS
    zrb = 160
    passes = [(o, min(zrb, zr - o)) for o in range(0, zr, zrb)]

    mesh = plsc.VectorSubcoreMesh(core_axis_name="c", subcore_axis_name="s")

    @functools.partial(
        pl.kernel,
        out_type=jax.ShapeDtypeStruct((NC, n_pad, dh), jnp.float32),
        mesh=mesh,
        scratch_types=(
            [pltpu.VMEM((nchunk * CHUNK,), jnp.int32),
             pltpu.VMEM((nchunk, CHUNK), jnp.int32)]
            + [pltpu.VMEM((CHUNK, dh), jnp.float32)] * NSLOT
            + [pltpu.VMEM((zrb, dh), jnp.float32)]
            + [pltpu.VMEM_SHARED((n_pad, dh), jnp.float32)]
            + [pltpu.SemaphoreType.DMA] * (2 * NSLOT)
        ),
        compiler_params=pltpu.CompilerParams(use_tc_tiling_on_sc=False),
    )
    def seg(feat_hbm, src_hbm, dst_hbm, out_hbm, srcb, dstb, *rest):
        rows = list(rest[:NSLOT])
        buf_v = rest[NSLOT]
        acc_sh = rest[NSLOT + 1]
        gsem = list(rest[NSLOT + 2:2 * NSLOT + 2])
        ssem = list(rest[2 * NSLOT + 2:])
        c = lax.axis_index("c")
        s = lax.axis_index("s")

        cp_src = pltpu.async_copy(src_hbm.at[c * NS + s], srcb, gsem[0])
        cp_dst = pltpu.async_copy(dst_hbm.at[s], dstb, gsem[1])

        def zrow(r, carry):
            for j in range(dh // 16):
                buf_v[r, pl.ds(j * 16, 16)] = jnp.zeros((16,), jnp.float32)
            return carry
        lax.fori_loop(0, zrb, zrow, 0)
        for off, sz in passes:
            pltpu.sync_copy(buf_v.at[pl.ds(0, sz)],
                            acc_sh.at[pl.ds(s * zr + off, sz)])
        cp_src.wait()
        cp_dst.wait()
        plsc.subcore_barrier()


        def issue_g(v, b):
            pltpu.async_copy(
                feat_hbm.at[srcb.at[pl.ds(v * CHUNK, CHUNK)]],
                rows[b], gsem[b])

        def wait_g(v, b):
            pltpu.make_async_copy(
                feat_hbm.at[srcb.at[pl.ds(v * CHUNK, CHUNK)]],
                rows[b], gsem[b]).wait()

        def issue_s(v, b):
            pltpu.async_copy(rows[b], acc_sh.at[dstb.at[v]], ssem[b],
                             add=True)

        def wait_s(v, b):
            pltpu.make_async_copy(rows[b], acc_sh.at[dstb.at[v]],
                                  ssem[b]).wait()

        for b in range(GLEAD):
            issue_g(b, b)

        def visitn(i, carry):
            for b in range(NSLOT):
                v = NSLOT * i + b

                @pl.when(v < nchunk)
                def _(v=v, b=b):
                    wait_g(v, b)
                    issue_s(v, b)

                @pl.when(jnp.logical_and(v >= SDEFER, v < nchunk + SDEFER))
                def _(v=v, b=b):
                    wait_s(v - SDEFER, (b - SDEFER) % NSLOT)

                @pl.when(v + GLEAD < nchunk)
                def _(v=v, b=b):
                    issue_g(v + GLEAD, (b + GLEAD) % NSLOT)
            return carry
        lax.fori_loop(0, (nchunk + NSLOT - 1) // NSLOT, visitn, 0)

        plsc.subcore_barrier()
        pltpu.sync_copy(acc_sh.at[pl.ds(s * zr, zr)],
                        out_hbm.at[c, pl.ds(s * zr, zr)])

    return seg(feat2, src2, dst3)


def _dense(parts, x, w_l, b_l, w_r, do_norm, planes_in, planes_out, n_pad, n):
    d = NC * parts.shape[2]
    rb = 1000
    dh = d // NC

    def body(p_ref, x_ref, wl_ref, b_ref, wr_ref, o_ref):
        agg = jnp.concatenate([p_ref[0], p_ref[1]], axis=1)
        dn = (((1,), (1,)), ((), ()))
        y = lax.dot_general(agg, wl_ref[...], dn,
                            preferred_element_type=jnp.float32)
        if planes_in:
            xcat = jnp.concatenate([x_ref[0], x_ref[1]], axis=1)
        else:
            xcat = x_ref[...]
        y = y + lax.dot_general(xcat, wr_ref[...], dn,
                                preferred_element_type=jnp.float32)
        y = y + b_ref[...]
        if do_norm:
            y = jnp.maximum(y, 0.0)
            nrm = jnp.sqrt(jnp.sum(y * y, axis=1, keepdims=True))
            y = y / jnp.maximum(nrm, 1e-12)
        if planes_out:
            o_ref[0] = y[:, :dh]
            o_ref[1] = y[:, dh:]
        else:
            o_ref[...] = y

    if planes_in:
        x_spec = pl.BlockSpec((NC, rb, dh), lambda i: (0, i, 0))
    else:
        x_spec = pl.BlockSpec((rb, d), lambda i: (i, 0))
    if planes_out:
        o_spec = pl.BlockSpec((NC, rb, dh), lambda i: (0, i, 0))
        o_shape = jax.ShapeDtypeStruct((NC, n_pad, dh), jnp.float32)
    else:
        o_spec = pl.BlockSpec((rb, d), lambda i: (i, 0))
        o_shape = jax.ShapeDtypeStruct((n, d), jnp.float32)

    return pl.pallas_call(
        body,
        grid=(n // rb,),
        in_specs=[
            pl.BlockSpec((NC, rb, dh), lambda i: (0, i, 0)),
            x_spec,
            pl.BlockSpec((d, d), lambda i: (0, 0)),
            pl.BlockSpec((1, d), lambda i: (0, 0)),
            pl.BlockSpec((d, d), lambda i: (0, 0)),
        ],
        out_specs=o_spec,
        out_shape=o_shape,
    )(parts, x, w_l, b_l.reshape(1, d), w_r)


def kernel(x, edge_index, edge_feature, W_l0, b_l0, W_r0, W_l1, b_l1, W_r1):
    n, d = x.shape
    dh = d // NC
    e = edge_index.shape[1]
    per_tile = e // NS
    nchunk = -(-per_tile // CHUNK)
    pt_pad = nchunk * CHUNK
    pad = pt_pad - per_tile
    zr = -(-n // (NS * 8)) * 8
    n_pad = zr * NS
    assert pad == 0, "edge count must tile evenly into CHUNK-size steps"
    src = edge_index[0]
    dst = edge_index[1]
    src2 = jnp.concatenate([src, n_pad + src]).reshape(NC * NS, per_tile)
    dst3 = dst.reshape(NS, nchunk, CHUNK)

    xpad = jnp.pad(x, ((0, n_pad - n), (0, 0)))
    x2p = jnp.stack([xpad[:, :dh], xpad[:, dh:]])

    p0 = _segment_sum_sc(x2p.reshape(NC * n_pad, dh), src2, dst3, n)
    h = _dense(p0, x, W_l0, b_l0, W_r0, do_norm=True,
               planes_in=False, planes_out=True, n_pad=n_pad, n=n)
    p1 = _segment_sum_sc(h.reshape(NC * n_pad, dh), src2, dst3, n)
    return _dense(p1, h, W_l1, b_l1, W_r1, do_norm=False,
                  planes_in=True, planes_out=False, n_pad=n_pad, n=n)


# speedup vs baseline: 1.1333x; 1.1333x over previous
import functools

import jax
import jax.numpy as jnp
from jax import lax
from jax.experimental import pallas as pl
from jax.experimental.pallas import tpu as pltpu
from jax.experimental.pallas import tpu_sc as plsc

NC = 2
NS = 16
CHUNK = 80
NSLOT = 4
GLEAD = 3
SDEFER = 1


def _segment_sum_sc(feat2, src2, dst3, n):
    dh = feat2.shape[1]
    nchunk = dst3.shape[1]
    npair = nchunk // 2
    zr = -(-n // (NS * 8)) * 8
    n_pad = zr * NS
    zrb = 160
    passes = [(o, min(zrb, zr - o)) for o in range(0, zr, zrb)]

    mesh = plsc.VectorSubcoreMesh(core_axis_name="c", subcore_axis_name="s")

    @functools.partial(
        pl.kernel,
        out_type=jax.ShapeDtypeStruct((NC, n_pad, dh), jnp.float32),
        mesh=mesh,
        scratch_types=(
            [pltpu.VMEM((nchunk * CHUNK,), jnp.int32),
             pltpu.VMEM((nchunk, CHUNK), jnp.int32)]
            + [pltpu.VMEM((CHUNK, dh), jnp.float32)] * NSLOT
            + [pltpu.VMEM((zrb, dh), jnp.float32)]
            + [pltpu.VMEM_SHARED((n_pad, dh), jnp.float32)]
            + [pltpu.SemaphoreType.DMA] * (2 * NSLOT)
        ),
        compiler_params=pltpu.CompilerParams(use_tc_tiling_on_sc=False),
    )
    def seg(feat_hbm, src_hbm, dst_hbm, out_hbm, srcb, dstb, *rest):
        rows = list(rest[:NSLOT])
        buf_v = rest[NSLOT]
        acc_sh = rest[NSLOT + 1]
        gsem = list(rest[NSLOT + 2:2 * NSLOT + 2])
        ssem = list(rest[2 * NSLOT + 2:])
        c = lax.axis_index("c")
        s = lax.axis_index("s")

        cp_src = pltpu.async_copy(src_hbm.at[c * NS + s], srcb, gsem[0])
        cp_dst = pltpu.async_copy(dst_hbm.at[s], dstb, gsem[1])

        def zrow(r, carry):
            for j in range(dh // 16):
                buf_v[r, pl.ds(j * 16, 16)] = jnp.zeros((16,), jnp.float32)
            return carry
        lax.fori_loop(0, zrb, zrow, 0)
        for off, sz in passes:
            pltpu.sync_copy(buf_v.at[pl.ds(0, sz)],
                            acc_sh.at[pl.ds(s * zr + off, sz)])
        cp_src.wait()
        cp_dst.wait()
        plsc.subcore_barrier()


        def issue_g(v, b):
            pltpu.async_copy(
                feat_hbm.at[srcb.at[pl.ds(v * CHUNK, CHUNK)]],
                rows[b], gsem[b])

        def wait_g(v, b):
            pltpu.make_async_copy(
                feat_hbm.at[srcb.at[pl.ds(v * CHUNK, CHUNK)]],
                rows[b], gsem[b]).wait()

        def issue_s(v, b):
            pltpu.async_copy(rows[b], acc_sh.at[dstb.at[v]], ssem[b],
                             add=True)

        def wait_s(v, b):
            pltpu.make_async_copy(rows[b], acc_sh.at[dstb.at[v]],
                                  ssem[b]).wait()

        for b in range(GLEAD):
            issue_g(b, b)

        def visitn(i, carry):
            for b in range(NSLOT):
                v = NSLOT * i + b

                @pl.when(v < nchunk)
                def _(v=v, b=b):
                    wait_g(v, b)
                    issue_s(v, b)

                @pl.when(jnp.logical_and(v >= SDEFER, v < nchunk + SDEFER))
                def _(v=v, b=b):
                    wait_s(v - SDEFER, (b - SDEFER) % NSLOT)

                @pl.when(v + GLEAD < nchunk)
                def _(v=v, b=b):
                    issue_g(v + GLEAD, (b + GLEAD) % NSLOT)
            return carry
        lax.fori_loop(0, (nchunk + NSLOT - 1) // NSLOT, visitn, 0)

        plsc.subcore_barrier()
        pltpu.sync_copy(acc_sh.at[pl.ds(s * zr, zr)],
                        out_hbm.at[c, pl.ds(s * zr, zr)])

    return seg(feat2, src2, dst3)


def _dense(parts, x, w_l, b_l, w_r, do_norm):
    n, d = x.shape
    rb = 1000
    dh = d // NC

    def body(p_ref, x_ref, wl_ref, b_ref, wr_ref, o_ref):
        agg = jnp.concatenate([p_ref[0], p_ref[1]], axis=1)
        dn = (((1,), (1,)), ((), ()))
        y = lax.dot_general(agg, wl_ref[...], dn,
                            preferred_element_type=jnp.float32)
        y = y + lax.dot_general(x_ref[...], wr_ref[...], dn,
                                preferred_element_type=jnp.float32)
        y = y + b_ref[...]
        if do_norm:
            y = jnp.maximum(y, 0.0)
            nrm = jnp.sqrt(jnp.sum(y * y, axis=1, keepdims=True))
            y = y / jnp.maximum(nrm, 1e-12)
        o_ref[...] = y

    return pl.pallas_call(
        body,
        grid=(n // rb,),
        in_specs=[
            pl.BlockSpec((NC, rb, dh), lambda i: (0, i, 0)),
            pl.BlockSpec((rb, d), lambda i: (i, 0)),
            pl.BlockSpec((d, d), lambda i: (0, 0)),
            pl.BlockSpec((1, d), lambda i: (0, 0)),
            pl.BlockSpec((d, d), lambda i: (0, 0)),
        ],
        out_specs=pl.BlockSpec((rb, d), lambda i: (i, 0)),
        out_shape=jax.ShapeDtypeStruct((n, d), jnp.float32),
    )(parts, x, w_l, b_l.reshape(1, d), w_r)


def kernel(x, edge_index, edge_feature, W_l0, b_l0, W_r0, W_l1, b_l1, W_r1):
    n, d = x.shape
    dh = d // NC
    e = edge_index.shape[1]
    per_tile = e // NS
    nchunk = -(-per_tile // CHUNK)
    pt_pad = nchunk * CHUNK
    pad = pt_pad - per_tile
    zr = -(-n // (NS * 8)) * 8
    n_pad = zr * NS
    src = edge_index[0]
    dst = edge_index[1]
    src2 = jnp.concatenate([2 * src, 2 * src + 1]).reshape(NC * NS, per_tile)
    src2 = jnp.pad(src2, ((0, 0), (0, pad)))
    dst3 = jnp.pad(dst.reshape(NS, per_tile), ((0, 0), (0, pad)),
                   constant_values=n_pad - 1).reshape(NS, nchunk, CHUNK)

    p0 = _segment_sum_sc(x.reshape(NC * n, dh), src2, dst3, n)
    h = _dense(p0, x, W_l0, b_l0, W_r0, do_norm=True)
    p1 = _segment_sum_sc(h.reshape(NC * n, dh), src2, dst3, n)
    return _dense(p1, h, W_l1, b_l1, W_r1, do_norm=False)
